# Initial kernel scaffold; baseline (speedup 1.0000x reference)
import jax
import jax.numpy as jnp
from jax.experimental import pallas as pl


def kernel(x, edge_index, W1, b1, W2, b2, W3, b3):
    raise NotImplementedError("write your pallas kernel here")


# speedup vs baseline: 7.7291x; 7.7291x over previous
import functools

import jax
import jax.numpy as jnp
from jax import lax
from jax.experimental import pallas as pl
from jax.experimental.pallas import tpu as pltpu
from jax.experimental.pallas import tpu_sc as plsc

N = 10000
E = 320000
F = 128
NC = 2
NS = 16
NW = NC * NS
RPT = 640
NPAD = NS * RPT
B = 128
NRB = RPT // B
NB = 79
EPT = NB * B
EPAD = NW * EPT

_MESH = dict(core_axis_name="c", subcore_axis_name="s")


def _make_agg_kernel(add_ones):
    @functools.partial(
        pl.kernel,
        out_type=jax.ShapeDtypeStruct((NC, NPAD, F), jnp.float32),
        mesh=plsc.VectorSubcoreMesh(**_MESH),
        scratch_types=[
            pltpu.VMEM((B,), jnp.int32),
            pltpu.VMEM((B,), jnp.int32),
            pltpu.VMEM((B, F), jnp.float32),
            pltpu.VMEM_SHARED((NPAD, F), jnp.float32),
            pltpu.SemaphoreType.DMA,
        ],
    )
    def agg_kernel(src_hbm, dst_hbm, g_hbm, fill_hbm, rowids_hbm, out_hbm,
                   sidx_v, didx_v, rows_v, acc_sh, sem):
        cid = lax.axis_index("c")
        sid = lax.axis_index("s")
        tid = cid * NS + sid
        pltpu.sync_copy(fill_hbm, rows_v)
        for j in range(NRB):
            rb = pl.ds(sid * RPT + j * B, B)
            pltpu.sync_copy(rowids_hbm.at[rb], sidx_v)
            pltpu.sync_copy(rows_v, acc_sh.at[sidx_v])
        if add_ones:
            pltpu.sync_copy(g_hbm.at[pl.ds(0, B)], rows_v)
        plsc.subcore_barrier()

        def body(b, carry):
            base = tid * EPT + b * B
            pltpu.sync_copy(dst_hbm.at[pl.ds(base, B)], didx_v)
            if not add_ones:
                pltpu.sync_copy(src_hbm.at[pl.ds(base, B)], sidx_v)
                pltpu.async_copy(g_hbm.at[sidx_v], rows_v, sem).wait()
            pltpu.sync_copy(rows_v, acc_sh.at[didx_v], add=True)
            return carry

        lax.fori_loop(0, NB, body, 0)
        plsc.subcore_barrier()
        for j in range(NRB):
            rb = pl.ds(sid * RPT + j * B, B)
            pltpu.sync_copy(rowids_hbm.at[rb], sidx_v)
            pltpu.async_copy(acc_sh.at[sidx_v], rows_v, sem).wait()
            pltpu.sync_copy(rows_v, out_hbm.at[cid, rb])

    return agg_kernel


_GRID = 50
_BR = N // _GRID


def _mm_body(x_ref, w_ref, o_ref):
    o_ref[...] = jnp.dot(x_ref[...], w_ref[...],
                         preferred_element_type=jnp.float32)


def _matmul(x, w):
    k = x.shape[1]
    n = w.shape[1]
    return pl.pallas_call(
        _mm_body,
        grid=(_GRID,),
        in_specs=[pl.BlockSpec((_BR, k), lambda i: (i, 0)),
                  pl.BlockSpec((k, n), lambda i: (0, 0))],
        out_specs=pl.BlockSpec((_BR, n), lambda i: (i, 0)),
        out_shape=jax.ShapeDtypeStruct((N, n), jnp.float32),
    )(x, w)


def _scale1_body(cnt_ref, h_ref, g_ref, dinv_ref):
    deg = cnt_ref[0][:, 0:1] + cnt_ref[1][:, 0:1] + 1.0
    dinv = lax.rsqrt(deg)
    dinv_ref[...] = dinv
    g_ref[...] = h_ref[...] * dinv


def _scale1(cnt, h):
    return pl.pallas_call(
        _scale1_body,
        grid=(_GRID,),
        in_specs=[pl.BlockSpec((NC, _BR, F), lambda i: (0, i, 0)),
                  pl.BlockSpec((_BR, F), lambda i: (i, 0))],
        out_specs=[pl.BlockSpec((_BR, F), lambda i: (i, 0)),
                   pl.BlockSpec((_BR, 1), lambda i: (i, 0))],
        out_shape=[jax.ShapeDtypeStruct((NPAD, F), jnp.float32),
                   jax.ShapeDtypeStruct((N, 1), jnp.float32)],
    )(cnt, h)


def _layer_body(a_ref, g_ref, dinv_ref, b_ref, w_ref, o_ref):
    dinv = dinv_ref[...]
    h = dinv * (a_ref[0] + a_ref[1] + g_ref[...]) + b_ref[...]
    h = jnp.maximum(h, 0.0)
    o_ref[...] = dinv * jnp.dot(h, w_ref[...],
                                preferred_element_type=jnp.float32)


def _layer(a, g, dinv, b, w):
    return pl.pallas_call(
        _layer_body,
        grid=(_GRID,),
        in_specs=[pl.BlockSpec((NC, _BR, F), lambda i: (0, i, 0)),
                  pl.BlockSpec((_BR, F), lambda i: (i, 0)),
                  pl.BlockSpec((_BR, 1), lambda i: (i, 0)),
                  pl.BlockSpec((1, F), lambda i: (0, 0)),
                  pl.BlockSpec((F, F), lambda i: (0, 0))],
        out_specs=pl.BlockSpec((_BR, F), lambda i: (i, 0)),
        out_shape=jax.ShapeDtypeStruct((NPAD, F), jnp.float32),
    )(a, g, dinv, b, w)


def _final_body(a_ref, g_ref, dinv_ref, b_ref, o_ref):
    h = dinv_ref[...] * (a_ref[0] + a_ref[1] + g_ref[...])
    h = h[:, :32] + b_ref[...]
    m = jnp.max(h, axis=1, keepdims=True)
    lse = jnp.log(jnp.sum(jnp.exp(h - m), axis=1, keepdims=True)) + m
    o_ref[...] = h - lse


def _final(a, g, dinv, b):
    return pl.pallas_call(
        _final_body,
        grid=(_GRID,),
        in_specs=[pl.BlockSpec((NC, _BR, F), lambda i: (0, i, 0)),
                  pl.BlockSpec((_BR, F), lambda i: (i, 0)),
                  pl.BlockSpec((_BR, 1), lambda i: (i, 0)),
                  pl.BlockSpec((1, 32), lambda i: (0, 0))],
        out_specs=pl.BlockSpec((_BR, 32), lambda i: (i, 0)),
        out_shape=jax.ShapeDtypeStruct((N, 32), jnp.float32),
    )(a, g, dinv, b)


def kernel(x, edge_index, W1, b1, W2, b2, W3, b3):
    ei = edge_index.astype(jnp.int32)
    pad = EPAD - E
    src = jnp.concatenate([ei[0], jnp.zeros((pad,), jnp.int32)])
    dst = jnp.concatenate([ei[1], jnp.full((pad,), N, jnp.int32)])

    zrows = jnp.zeros((B, F), jnp.float32)
    ones_tab = jnp.ones((B, F), jnp.float32)
    rowids = jnp.arange(NPAD, dtype=jnp.int32)

    agg = _make_agg_kernel(add_ones=False)
    deg = _make_agg_kernel(add_ones=True)

    cnt = deg(src, dst, ones_tab, zrows, rowids)
    h1 = _matmul(x, W1)
    g1, dinv = _scale1(cnt, h1)

    W2p = jnp.pad(W2, ((0, 0), (0, F - W2.shape[1])))
    W3p = jnp.pad(W3, ((0, F - W3.shape[0]), (0, F - W3.shape[1])))
    b1p = b1.reshape(1, -1)
    b2p = jnp.pad(b2, (0, F - b2.shape[0])).reshape(1, -1)

    a1 = agg(src, dst, g1, zrows, rowids)
    g2 = _layer(a1, g1, dinv, b1p, W2p)

    a2 = agg(src, dst, g2, zrows, rowids)
    g3 = _layer(a2, g2, dinv, b2p, W3p)

    a3 = agg(src, dst, g3, zrows, rowids)
    return _final(a3, g3, dinv, b3.reshape(1, -1))
